# baseline (speedup 1.0000x reference)
import functools

import jax
import jax.numpy as jnp
from jax import lax
from jax.experimental import pallas as pl
from jax.experimental.pallas import tpu as pltpu


def _chamfer_body(x_ref, y_ref, rsum_ref, cmin_ref, *, tn):
    n = x_ref.shape[1]
    m = y_ref.shape[1]

    ym = y_ref[0]
    y2c = jnp.sum(ym * ym, axis=1, keepdims=True)
    y2 = y2c.reshape(1, m)

    racc = None
    cacc = None
    for i in range(n // tn):
        xa = x_ref[0, pl.ds(i * tn, tn), :]
        x2 = jnp.sum(xa * xa, axis=1, keepdims=True)
        t = lax.dot_general(xa * -2.0, ym, (((1,), (1,)), ((), ())),
                            preferred_element_type=jnp.float32)

        rmin = jnp.min(t + y2, axis=1, keepdims=True) + x2
        rclamp = jnp.maximum(rmin, 0.0)
        racc = rclamp if racc is None else racc + rclamp

        vfold = jnp.min((t + x2).reshape(tn // 8, 8, m), axis=0)
        cacc = vfold if cacc is None else jnp.minimum(cacc, vfold)

    rsum_ref[...] = jnp.sum(racc).reshape(1, 1, 1)
    cm = jnp.min(cacc, axis=0, keepdims=True)
    cmin_ref[0] = jnp.maximum(cm + y2, 0.0)


@functools.partial(jax.jit, static_argnames=("tn",))
def _chamfer(x, y, tn=1024):
    b, n, _ = x.shape
    m = y.shape[1]

    rsums, cmins = pl.pallas_call(
        functools.partial(_chamfer_body, tn=tn),
        grid=(b,),
        in_specs=[
            pl.BlockSpec((1, n, 3), lambda bi: (bi, 0, 0)),
            pl.BlockSpec((1, m, 3), lambda bi: (bi, 0, 0)),
        ],
        out_specs=[
            pl.BlockSpec((1, 1, 1), lambda bi: (bi, 0, 0)),
            pl.BlockSpec((1, 1, m), lambda bi: (bi, 0, 0)),
        ],
        out_shape=[
            jax.ShapeDtypeStruct((b, 1, 1), jnp.float32),
            jax.ShapeDtypeStruct((b, 1, m), jnp.float32),
        ],
        compiler_params=pltpu.CompilerParams(
            dimension_semantics=("parallel",)),
    )(x, y)

    x_to_y = jnp.mean(rsums) / n
    y_to_x = jnp.mean(cmins)
    return jnp.maximum(x_to_y, y_to_x)


def kernel(x, y):
    return _chamfer(x, y)


# speedup vs baseline: 1.0522x; 1.0522x over previous
import functools

import jax
import jax.numpy as jnp
from jax import lax
from jax.experimental import pallas as pl
from jax.experimental.pallas import tpu as pltpu


def _chamfer_body(x_ref, y_ref, rsum_ref, cmin_ref, *, tn):
    n = x_ref.shape[1]
    m = y_ref.shape[1]

    ym = y_ref[0]
    y2c = jnp.sum(ym * ym, axis=1, keepdims=True)
    y2 = y2c.reshape(1, m)

    racc = None
    cacc = None
    for i in range(n // tn):
        xa = x_ref[0, pl.ds(i * tn, tn), :]
        x2 = jnp.sum(xa * xa, axis=1, keepdims=True)
        t = lax.dot_general(xa * -2.0, ym, (((1,), (1,)), ((), ())),
                            preferred_element_type=jnp.float32)

        rmin = jnp.min(t + y2, axis=1, keepdims=True) + x2
        rclamp = jnp.maximum(rmin, 0.0)
        racc = rclamp if racc is None else racc + rclamp

        vfold = jnp.min((t + x2).reshape(tn // 8, 8, m), axis=0)
        cacc = vfold if cacc is None else jnp.minimum(cacc, vfold)

    rsum_ref[...] = jnp.sum(racc).reshape(1, 1, 1)
    cm = jnp.min(cacc, axis=0, keepdims=True)
    cmin_ref[0] = jnp.maximum(cm + y2, 0.0)


@functools.partial(jax.jit, static_argnames=("tn",))
def _chamfer(x, y, tn=256):
    b, n, _ = x.shape
    m = y.shape[1]

    rsums, cmins = pl.pallas_call(
        functools.partial(_chamfer_body, tn=tn),
        grid=(b,),
        in_specs=[
            pl.BlockSpec((1, n, 3), lambda bi: (bi, 0, 0)),
            pl.BlockSpec((1, m, 3), lambda bi: (bi, 0, 0)),
        ],
        out_specs=[
            pl.BlockSpec((1, 1, 1), lambda bi: (bi, 0, 0)),
            pl.BlockSpec((1, 1, m), lambda bi: (bi, 0, 0)),
        ],
        out_shape=[
            jax.ShapeDtypeStruct((b, 1, 1), jnp.float32),
            jax.ShapeDtypeStruct((b, 1, m), jnp.float32),
        ],
        compiler_params=pltpu.CompilerParams(
            dimension_semantics=("parallel",)),
    )(x, y)

    x_to_y = jnp.mean(rsums) / n
    y_to_x = jnp.mean(cmins)
    return jnp.maximum(x_to_y, y_to_x)


def kernel(x, y):
    return _chamfer(x, y)
